# baseline (speedup 1.0000x reference)
import functools

import jax
import jax.numpy as jnp
from jax import lax
from jax.experimental import pallas as pl
from jax.experimental.pallas import tpu as pltpu
from jax.experimental.pallas import tpu_sc as plsc

N = 10000
E = 320000
B = 4
F = 128

NC = 2
NS = 16
NW = NC * NS
L = 16

EPT_H = E // NW

EPT = E // NS
CH = 128
GC = 4
EPT_P = 20480
NG = EPT_P // (GC * CH)
QLO = (0, 2512, 5024, 7536)
QSZ = (2512, 2512, 2512, 2464)
AOFF = 2520
ACC_R = 2 * AOFF
PLI = 20608
CH2 = 64
ZS = 312
ZTAIL = ACC_R - NS * ZS
DS = 152


def _hist_body(row_hbm, out_hbm, idx_v, hist_v, sem):
    c = lax.axis_index("c")
    s = lax.axis_index("s")
    wid = s * NC + c
    pltpu.sync_copy(row_hbm.at[pl.ds(wid * EPT_H, EPT_H)], idx_v)

    def zero(i, _):
        hist_v[pl.ds(i * L, L)] = jnp.zeros((L,), jnp.float32)
        return 0

    lax.fori_loop(0, N // L, zero, 0)
    ones = jnp.ones((L,), jnp.float32)

    def scat(i, _):
        iv = idx_v[pl.ds(i * L, L)]
        plsc.addupdate_scatter(hist_v, [iv], ones)
        return 0

    lax.fori_loop(0, EPT_H // L, scat, 0)
    pltpu.sync_copy(hist_v, out_hbm.at[wid])


def _sc_hist(row32):
    mesh = plsc.VectorSubcoreMesh(
        core_axis_name="c", subcore_axis_name="s", num_cores=NC,
        num_subcores=NS)
    f = pl.kernel(
        _hist_body,
        out_type=jax.ShapeDtypeStruct((NW, N), jnp.float32),
        mesh=mesh,
        compiler_params=pltpu.CompilerParams(needs_layout_passes=False),
        scratch_types=[
            pltpu.VMEM((EPT_H,), jnp.int32),
            pltpu.VMEM((N,), jnp.float32),
            pltpu.SemaphoreType.DMA,
        ],
    )
    return f(row32)


def _dis_of(hist_blk):
    deg = jnp.sum(hist_blk, axis=1) + 1.0
    return lax.rsqrt(deg)


def _lin_body(hist_ref, x_ref, w_ref, b_ref, y2_ref):
    dis = _dis_of(hist_ref[...])
    w = w_ref[...]
    bias = b_ref[...]
    for bi in range(B):
        xl = lax.dot_general(
            x_ref[bi], w, (((1,), (1,)), ((), ())),
            preferred_element_type=jnp.float32)
        y2_ref[bi // 2, :, (bi % 2) * F:(bi % 2 + 1) * F] = (
            dis[:, None] * (xl + bias))


def _tc_linear(hist_t, x_batch, W, bias):
    BLK = 1000
    G = N // BLK
    return pl.pallas_call(
        _lin_body,
        grid=(G,),
        in_specs=[
            pl.BlockSpec((BLK, NW), lambda i: (i, 0)),
            pl.BlockSpec((B, BLK, F), lambda i: (0, i, 0)),
            pl.BlockSpec((F, F), lambda i: (0, 0)),
            pl.BlockSpec((1, F), lambda i: (0, 0)),
        ],
        out_specs=pl.BlockSpec((NC, BLK, 2 * F), lambda i: (0, i, 0)),
        out_shape=jax.ShapeDtypeStruct((NC, N, 2 * F), jnp.float32),
    )(hist_t, x_batch, W, bias)


def _spmm_body(row_hbm, col_hbm, y_hbm, out_hbm,
               rbufg, cbufg, plist, gbuf, hb0, hb1, cidx, ridx, acc_sh,
               i0, i1, g0, g1, s0, s1):
    c = lax.axis_index("c")
    s = lax.axis_index("s")
    isems = (i0, i1)
    gsems = (g0, g1)
    ssems = (s0, s1)

    def zfill_hb0():
        def zf(i, _):
            hb0[0, i // (F // L), pl.ds((i % (F // L)) * L, L)] = (
                jnp.zeros((L,), jnp.float32))
            return 0

        lax.fori_loop(0, CH2 * (F // L), zf, 0)

    def stage_idx(g, a):
        pltpu.async_copy(row_hbm.at[s].at[g], rbufg.at[a], isems[a])
        pltpu.async_copy(col_hbm.at[s].at[g], cbufg.at[a], isems[a])

    def wait_idx(a):
        pltpu.make_async_copy(row_hbm.at[s].at[0], rbufg.at[a],
                              isems[a]).wait()
        pltpu.make_async_copy(row_hbm.at[s].at[0], cbufg.at[a],
                              isems[a]).wait()

    cbase = c * N

    def unpack_cidx(j, buf):
        for t in range(CH2 // L):
            v = plist[pl.ds(j * CH2 + t * L, L)]
            cidx[buf, pl.ds(t * L, L)] = (
                jax.lax.shift_right_logical(v, 14) + cbase)

    def gather(j, buf):
        pltpu.async_copy(y_hbm.at[cidx.at[buf]], gbuf.at[buf], gsems[buf])

    def wait_gather(buf):
        pltpu.make_async_copy(y_hbm.at[pl.ds(0, CH2)], gbuf.at[buf],
                              gsems[buf]).wait()

    base_z = pl.multiple_of(s * ZS, 8)
    base_d = pl.multiple_of(s * DS, 8)

    for p in range(4):
        lo = QLO[p]
        sz = QSZ[p]

        zfill_hb0()
        for j in range(ZS // CH2):
            pltpu.sync_copy(hb0.at[0],
                            acc_sh.at[pl.ds(base_z + j * CH2, CH2)])
        rem = ZS - (ZS // CH2) * CH2
        if rem:
            pltpu.sync_copy(hb0.at[0].at[pl.ds(0, rem)],
                            acc_sh.at[pl.ds(base_z + ZS - rem, rem)])

        @pl.when(s == 0)
        def _():
            pltpu.sync_copy(hb0.at[0].at[pl.ds(0, ZTAIL)],
                            acc_sh.at[pl.ds(NS * ZS, ZTAIL)])

        plsc.subcore_barrier()

        stage_idx(0, 0)
        stage_idx(1, 1)

        def gbody(gi, cnt):
            for a in range(2):
                g = gi * 2 + a
                wait_idx(a)

                def fvec(vv, cn):
                    jj = vv // (CH // L)
                    kk = vv % (CH // L)
                    r = rbufg[a, jj, pl.ds(kk * L, L)]
                    cl = cbufg[a, jj, pl.ds(kk * L, L)]
                    m = (r >= lo) & (r < lo + sz)
                    packed = jax.lax.bitwise_or(
                        r - lo, jax.lax.shift_left(cl, 14))
                    plsc.store_compressed(plist.at[pl.ds(cn, L)], packed,
                                          mask=m)
                    return cn + plsc.all_reduce_population_count(m)[0]

                cnt = lax.fori_loop(0, GC * CH // L, fvec, cnt)

                @pl.when(g + 2 < NG)
                def _():
                    stage_idx(g + 2, a)
            return cnt

        cnt = lax.fori_loop(0, NG // 2, gbody, 0)

        absv = jnp.full((L,), 2512, jnp.int32)
        for t in range(2 * CH2 // L):
            plist[pl.ds(cnt + t * L, L)] = absv
        npairs = lax.max((cnt + 2 * CH2 - 1) // (2 * CH2), 1)

        unpack_cidx(0, 0)
        gather(0, 0)
        unpack_cidx(1, 1)
        gather(1, 1)

        def wait_scats(buf):
            pltpu.make_async_copy(y_hbm.at[pl.ds(0, CH2 // 2)],
                                  hb0.at[buf], ssems[buf]).wait()
            pltpu.make_async_copy(y_hbm.at[pl.ds(0, CH2 // 2)],
                                  hb1.at[buf], ssems[buf]).wait()

        def pbody(k, _):
            for buf in range(2):
                j = 2 * k + buf
                wait_gather(buf)

                @pl.when(k > 0)
                def _():
                    wait_scats(buf)

                def crow(r):
                    for t in range(F // L):
                        hb0[buf, r, pl.ds(t * L, L)] = (
                            gbuf[buf, r, pl.ds(t * L, L)])
                        hb1[buf, r, pl.ds(t * L, L)] = (
                            gbuf[buf, r, pl.ds(F + t * L, L)])

                plsc.parallel_loop(0, CH2)(crow)

                @pl.when(k + 1 < npairs)
                def _():
                    unpack_cidx(j + 2, buf)
                    gather(j + 2, buf)

                for t in range(CH2 // L):
                    v = plist[pl.ds(j * CH2 + t * L, L)]
                    rv = jax.lax.bitwise_and(v, 16383)
                    ridx[2 * buf, pl.ds(t * L, L)] = rv
                    ridx[2 * buf + 1, pl.ds(t * L, L)] = rv + AOFF
                pltpu.async_copy(hb0.at[buf], acc_sh.at[ridx.at[2 * buf]],
                                 ssems[buf], add=True)
                pltpu.async_copy(hb1.at[buf],
                                 acc_sh.at[ridx.at[2 * buf + 1]],
                                 ssems[buf], add=True)
            return 0

        lax.fori_loop(0, npairs, pbody, 0)
        wait_scats(0)
        wait_scats(1)
        plsc.subcore_barrier()

        for half in range(2):
            fb = 2 * c + half
            ab = half * AOFF
            pltpu.sync_copy(
                acc_sh.at[pl.ds(ab + base_d, DS)],
                out_hbm.at[fb].at[pl.ds(lo + base_d, DS)])
            tail = sz - NS * DS

            @pl.when(s == 0)
            def _():
                pltpu.sync_copy(
                    acc_sh.at[pl.ds(ab + NS * DS, tail)],
                    out_hbm.at[fb].at[pl.ds(lo + NS * DS, tail)])

        plsc.subcore_barrier()


def _sc_spmm(row3, col3, y2):
    mesh = plsc.VectorSubcoreMesh(
        core_axis_name="c", subcore_axis_name="s", num_cores=NC,
        num_subcores=NS)
    f = pl.kernel(
        _spmm_body,
        out_type=jax.ShapeDtypeStruct((B, N, F), jnp.float32),
        mesh=mesh,
        compiler_params=pltpu.CompilerParams(needs_layout_passes=False),
        scratch_types=[
            pltpu.VMEM((2, GC, CH), jnp.int32),
            pltpu.VMEM((2, GC, CH), jnp.int32),
            pltpu.VMEM((PLI,), jnp.int32),
            pltpu.VMEM((2, CH2, 2 * F), jnp.float32),
            pltpu.VMEM((2, CH2, F), jnp.float32),
            pltpu.VMEM((2, CH2, F), jnp.float32),
            pltpu.VMEM((2, CH2), jnp.int32),
            pltpu.VMEM((4, CH2), jnp.int32),
            pltpu.VMEM_SHARED((ACC_R, F), jnp.float32),
            pltpu.SemaphoreType.DMA,
            pltpu.SemaphoreType.DMA,
            pltpu.SemaphoreType.DMA,
            pltpu.SemaphoreType.DMA,
            pltpu.SemaphoreType.DMA,
            pltpu.SemaphoreType.DMA,
        ],
    )
    return f(row3, col3, y2)


def _fin_body(acc_ref, y_ref, hist_ref, out_ref):
    dis = _dis_of(hist_ref[...])
    out_ref[...] = jnp.maximum(
        dis[None, :, None] * (acc_ref[...] + y_ref[...]), 0.0)


def _tc_final(acc4, y2, hist_t):
    BLK = 1000
    G = N // BLK
    return pl.pallas_call(
        _fin_body,
        grid=(B, G),
        in_specs=[
            pl.BlockSpec((1, BLK, F), lambda b, i: (b, i, 0)),
            pl.BlockSpec((1, BLK, F), lambda b, i: (b // 2, i, b % 2)),
            pl.BlockSpec((BLK, NW), lambda b, i: (i, 0)),
        ],
        out_specs=pl.BlockSpec((1, BLK, F), lambda b, i: (b, i, 0)),
        out_shape=jax.ShapeDtypeStruct((B, N, F), jnp.float32),
    )(acc4, y2, hist_t)


def kernel(x_batch, edge_index, W, b):
    ei = edge_index.astype(jnp.int32)
    row = ei[0]
    col = ei[1]
    hist_t = _sc_hist(row).T
    y2 = _tc_linear(hist_t, x_batch, W, b.reshape(1, F))
    pad = EPT_P - EPT
    row3 = jnp.concatenate(
        [row.reshape(NS, EPT),
         jnp.full((NS, pad), N, jnp.int32)], axis=1,
    ).reshape(NS, NG, GC, CH)
    col3 = jnp.concatenate(
        [col.reshape(NS, EPT),
         jnp.zeros((NS, pad), jnp.int32)], axis=1,
    ).reshape(NS, NG, GC, CH)
    acc4 = _sc_spmm(row3, col3, y2.reshape(NC * N, 2 * F))
    return _tc_final(acc4, y2, hist_t)


# speedup vs baseline: 1.0270x; 1.0270x over previous
import functools

import jax
import jax.numpy as jnp
from jax import lax
from jax.experimental import pallas as pl
from jax.experimental.pallas import tpu as pltpu
from jax.experimental.pallas import tpu_sc as plsc

N = 10000
E = 320000
B = 4
F = 128

NC = 2
NS = 16
NW = NC * NS
L = 16

EPT_H = E // NW

EPT = E // NS
CH = 128
GC = 4
EPT_P = 20480
NG = EPT_P // (GC * CH)
QLO = (0, 2512, 5024, 7536)
QSZ = (2512, 2512, 2512, 2464)
AOFF = 2520
ACC_R = 2 * AOFF
PLI = 20608
CH2 = 32
NBUF = 4
ZS = 312
ZTAIL = ACC_R - NS * ZS
DS = 152


def _hist_body(row_hbm, out_hbm, idx_v, hist_v, sem):
    c = lax.axis_index("c")
    s = lax.axis_index("s")
    wid = s * NC + c
    pltpu.sync_copy(row_hbm.at[pl.ds(wid * EPT_H, EPT_H)], idx_v)

    def zero(i, _):
        hist_v[pl.ds(i * L, L)] = jnp.zeros((L,), jnp.float32)
        return 0

    lax.fori_loop(0, N // L, zero, 0)
    ones = jnp.ones((L,), jnp.float32)

    def scat(i, _):
        iv = idx_v[pl.ds(i * L, L)]
        plsc.addupdate_scatter(hist_v, [iv], ones)
        return 0

    lax.fori_loop(0, EPT_H // L, scat, 0)
    pltpu.sync_copy(hist_v, out_hbm.at[wid])


def _sc_hist(row32):
    mesh = plsc.VectorSubcoreMesh(
        core_axis_name="c", subcore_axis_name="s", num_cores=NC,
        num_subcores=NS)
    f = pl.kernel(
        _hist_body,
        out_type=jax.ShapeDtypeStruct((NW, N), jnp.float32),
        mesh=mesh,
        compiler_params=pltpu.CompilerParams(needs_layout_passes=False),
        scratch_types=[
            pltpu.VMEM((EPT_H,), jnp.int32),
            pltpu.VMEM((N,), jnp.float32),
            pltpu.SemaphoreType.DMA,
        ],
    )
    return f(row32)


def _dis_of(hist_blk):
    deg = jnp.sum(hist_blk, axis=1) + 1.0
    return lax.rsqrt(deg)


def _lin_body(hist_ref, x_ref, w_ref, b_ref, y2_ref):
    dis = _dis_of(hist_ref[...])
    w = w_ref[...]
    bias = b_ref[...]
    for bi in range(B):
        xl = lax.dot_general(
            x_ref[bi], w, (((1,), (1,)), ((), ())),
            preferred_element_type=jnp.float32)
        y2_ref[bi // 2, :, (bi % 2) * F:(bi % 2 + 1) * F] = (
            dis[:, None] * (xl + bias))


def _tc_linear(hist_t, x_batch, W, bias):
    BLK = 1000
    G = N // BLK
    return pl.pallas_call(
        _lin_body,
        grid=(G,),
        in_specs=[
            pl.BlockSpec((BLK, NW), lambda i: (i, 0)),
            pl.BlockSpec((B, BLK, F), lambda i: (0, i, 0)),
            pl.BlockSpec((F, F), lambda i: (0, 0)),
            pl.BlockSpec((1, F), lambda i: (0, 0)),
        ],
        out_specs=pl.BlockSpec((NC, BLK, 2 * F), lambda i: (0, i, 0)),
        out_shape=jax.ShapeDtypeStruct((NC, N, 2 * F), jnp.float32),
    )(hist_t, x_batch, W, bias)


def _spmm_body(row_hbm, col_hbm, y_hbm, out_hbm,
               rbufg, cbufg, plist, gbuf, hb0, hb1, cidx, ridx, acc_sh,
               i0, i1, g0, g1, g2, g3, s0, s1, s2, s3):
    c = lax.axis_index("c")
    s = lax.axis_index("s")
    isems = (i0, i1)
    gsems = (g0, g1, g2, g3)
    ssems = (s0, s1, s2, s3)

    def zfill_hb0():
        def zf(i, _):
            hb0[0, i // (F // L), pl.ds((i % (F // L)) * L, L)] = (
                jnp.zeros((L,), jnp.float32))
            return 0

        lax.fori_loop(0, CH2 * (F // L), zf, 0)

    def stage_idx(g, a):
        pltpu.async_copy(row_hbm.at[s].at[g], rbufg.at[a], isems[a])
        pltpu.async_copy(col_hbm.at[s].at[g], cbufg.at[a], isems[a])

    def wait_idx(a):
        pltpu.make_async_copy(row_hbm.at[s].at[0], rbufg.at[a],
                              isems[a]).wait()
        pltpu.make_async_copy(row_hbm.at[s].at[0], cbufg.at[a],
                              isems[a]).wait()

    cbase = c * N

    def unpack_cidx(j, buf):
        for t in range(CH2 // L):
            v = plist[pl.ds(j * CH2 + t * L, L)]
            cidx[buf, pl.ds(t * L, L)] = (
                jax.lax.shift_right_logical(v, 14) + cbase)

    def gather(j, buf):
        pltpu.async_copy(y_hbm.at[cidx.at[buf]], gbuf.at[buf], gsems[buf])

    def wait_gather(buf):
        pltpu.make_async_copy(y_hbm.at[pl.ds(0, CH2)], gbuf.at[buf],
                              gsems[buf]).wait()

    base_z = pl.multiple_of(s * ZS, 8)
    base_d = pl.multiple_of(s * DS, 8)

    for p in range(4):
        lo = QLO[p]
        sz = QSZ[p]

        zfill_hb0()
        for j in range(ZS // CH2):
            pltpu.sync_copy(hb0.at[0],
                            acc_sh.at[pl.ds(base_z + j * CH2, CH2)])
        rem = ZS - (ZS // CH2) * CH2
        if rem:
            pltpu.sync_copy(hb0.at[0].at[pl.ds(0, rem)],
                            acc_sh.at[pl.ds(base_z + ZS - rem, rem)])

        @pl.when(s == 0)
        def _():
            pltpu.sync_copy(hb0.at[0],
                            acc_sh.at[pl.ds(NS * ZS, CH2)])
            pltpu.sync_copy(hb0.at[0].at[pl.ds(0, ZTAIL - CH2)],
                            acc_sh.at[pl.ds(NS * ZS + CH2, ZTAIL - CH2)])

        plsc.subcore_barrier()

        stage_idx(0, 0)
        stage_idx(1, 1)

        def gbody(gi, cnt):
            for a in range(2):
                g = gi * 2 + a
                wait_idx(a)

                def fvec(vv, cn):
                    jj = vv // (CH // L)
                    kk = vv % (CH // L)
                    r = rbufg[a, jj, pl.ds(kk * L, L)]
                    cl = cbufg[a, jj, pl.ds(kk * L, L)]
                    m = (r >= lo) & (r < lo + sz)
                    packed = jax.lax.bitwise_or(
                        r - lo, jax.lax.shift_left(cl, 14))
                    plsc.store_compressed(plist.at[pl.ds(cn, L)], packed,
                                          mask=m)
                    return cn + plsc.all_reduce_population_count(m)[0]

                cnt = lax.fori_loop(0, GC * CH // L, fvec, cnt)

                @pl.when(g + 2 < NG)
                def _():
                    stage_idx(g + 2, a)
            return cnt

        cnt = lax.fori_loop(0, NG // 2, gbody, 0)

        absv = jnp.full((L,), 2512, jnp.int32)
        for t in range(NBUF * CH2 // L):
            plist[pl.ds(cnt + t * L, L)] = absv
        npairs = lax.max((cnt + NBUF * CH2 - 1) // (NBUF * CH2), 1)

        for buf in range(NBUF):
            unpack_cidx(buf, buf)
            gather(buf, buf)

        def wait_scats(buf):
            pltpu.make_async_copy(y_hbm.at[pl.ds(0, CH2 // 2)],
                                  hb0.at[buf], ssems[buf]).wait()
            pltpu.make_async_copy(y_hbm.at[pl.ds(0, CH2 // 2)],
                                  hb1.at[buf], ssems[buf]).wait()

        def pbody(k, _):
            for buf in range(NBUF):
                j = NBUF * k + buf
                wait_gather(buf)

                @pl.when(k > 0)
                def _():
                    wait_scats(buf)

                def crow(r):
                    for t in range(F // L):
                        hb0[buf, r, pl.ds(t * L, L)] = (
                            gbuf[buf, r, pl.ds(t * L, L)])
                        hb1[buf, r, pl.ds(t * L, L)] = (
                            gbuf[buf, r, pl.ds(F + t * L, L)])

                plsc.parallel_loop(0, CH2, unroll=2)(crow)

                @pl.when(k + 1 < npairs)
                def _():
                    unpack_cidx(j + NBUF, buf)
                    gather(j + NBUF, buf)

                for t in range(CH2 // L):
                    v = plist[pl.ds(j * CH2 + t * L, L)]
                    rv = jax.lax.bitwise_and(v, 16383)
                    ridx[2 * buf, pl.ds(t * L, L)] = rv
                    ridx[2 * buf + 1, pl.ds(t * L, L)] = rv + AOFF
                pltpu.async_copy(hb0.at[buf], acc_sh.at[ridx.at[2 * buf]],
                                 ssems[buf], add=True)
                pltpu.async_copy(hb1.at[buf],
                                 acc_sh.at[ridx.at[2 * buf + 1]],
                                 ssems[buf], add=True)
            return 0

        lax.fori_loop(0, npairs, pbody, 0)
        for buf in range(NBUF):
            wait_scats(buf)
        plsc.subcore_barrier()

        for half in range(2):
            fb = 2 * c + half
            ab = half * AOFF
            pltpu.sync_copy(
                acc_sh.at[pl.ds(ab + base_d, DS)],
                out_hbm.at[fb].at[pl.ds(lo + base_d, DS)])
            tail = sz - NS * DS

            @pl.when(s == 0)
            def _():
                pltpu.sync_copy(
                    acc_sh.at[pl.ds(ab + NS * DS, tail)],
                    out_hbm.at[fb].at[pl.ds(lo + NS * DS, tail)])

        plsc.subcore_barrier()


def _sc_spmm(row3, col3, y2):
    mesh = plsc.VectorSubcoreMesh(
        core_axis_name="c", subcore_axis_name="s", num_cores=NC,
        num_subcores=NS)
    f = pl.kernel(
        _spmm_body,
        out_type=jax.ShapeDtypeStruct((B, N, F), jnp.float32),
        mesh=mesh,
        compiler_params=pltpu.CompilerParams(needs_layout_passes=False),
        scratch_types=[
            pltpu.VMEM((2, GC, CH), jnp.int32),
            pltpu.VMEM((2, GC, CH), jnp.int32),
            pltpu.VMEM((PLI,), jnp.int32),
            pltpu.VMEM((NBUF, CH2, 2 * F), jnp.float32),
            pltpu.VMEM((NBUF, CH2, F), jnp.float32),
            pltpu.VMEM((NBUF, CH2, F), jnp.float32),
            pltpu.VMEM((NBUF, CH2), jnp.int32),
            pltpu.VMEM((2 * NBUF, CH2), jnp.int32),
            pltpu.VMEM_SHARED((ACC_R, F), jnp.float32),
            pltpu.SemaphoreType.DMA,
            pltpu.SemaphoreType.DMA,
            pltpu.SemaphoreType.DMA,
            pltpu.SemaphoreType.DMA,
            pltpu.SemaphoreType.DMA,
            pltpu.SemaphoreType.DMA,
            pltpu.SemaphoreType.DMA,
            pltpu.SemaphoreType.DMA,
            pltpu.SemaphoreType.DMA,
            pltpu.SemaphoreType.DMA,
        ],
    )
    return f(row3, col3, y2)


def _fin_body(acc_ref, y_ref, hist_ref, out_ref):
    dis = _dis_of(hist_ref[...])
    out_ref[...] = jnp.maximum(
        dis[None, :, None] * (acc_ref[...] + y_ref[...]), 0.0)


def _tc_final(acc4, y2, hist_t):
    BLK = 1000
    G = N // BLK
    return pl.pallas_call(
        _fin_body,
        grid=(B, G),
        in_specs=[
            pl.BlockSpec((1, BLK, F), lambda b, i: (b, i, 0)),
            pl.BlockSpec((1, BLK, F), lambda b, i: (b // 2, i, b % 2)),
            pl.BlockSpec((BLK, NW), lambda b, i: (i, 0)),
        ],
        out_specs=pl.BlockSpec((1, BLK, F), lambda b, i: (b, i, 0)),
        out_shape=jax.ShapeDtypeStruct((B, N, F), jnp.float32),
    )(acc4, y2, hist_t)


def kernel(x_batch, edge_index, W, b):
    ei = edge_index.astype(jnp.int32)
    row = ei[0]
    col = ei[1]
    hist_t = _sc_hist(row).T
    y2 = _tc_linear(hist_t, x_batch, W, b.reshape(1, F))
    pad = EPT_P - EPT
    row3 = jnp.concatenate(
        [row.reshape(NS, EPT),
         jnp.full((NS, pad), N, jnp.int32)], axis=1,
    ).reshape(NS, NG, GC, CH)
    col3 = jnp.concatenate(
        [col.reshape(NS, EPT),
         jnp.zeros((NS, pad), jnp.int32)], axis=1,
    ).reshape(NS, NG, GC, CH)
    acc4 = _sc_spmm(row3, col3, y2.reshape(NC * N, 2 * F))
    return _tc_final(acc4, y2, hist_t)
